# baseline (speedup 1.0000x reference)
import jax
import jax.numpy as jnp
import numpy as np
from jax import lax
from jax.experimental import pallas as pl
from jax.experimental.pallas import tpu as pltpu
from jax.experimental.pallas import tpu_sc as plsc

_THETA = ((3.0, -3.0, 0.75), (0.0, 3.0, -1.5), (0.0, 0.0, 0.75))

_N = 10000
_NP = 10240
_E = 320000
_CHUNK = 128
_NCHUNK = 80
_NMAIN = 78
_NEXTRA = 2
_RPS = _NP // 16
_DH = 64

_BLK = 2000
_GRID = _N // _BLK

_NPAD_E = 64 * _CHUNK - 512
_PAD_SRC = np.asarray((np.arange(_NPAD_E) * 97) % _N, np.int32)
_PAD_DST = np.asarray(_N + np.arange(_NPAD_E) % (_NP - _N), np.int32)


def _mesh():
    return plsc.VectorSubcoreMesh(core_axis_name="c", subcore_axis_name="s")


_SC_PARAMS = pltpu.CompilerParams(use_tc_tiling_on_sc=False)


def _load_idx(main2d, extra2d, w, dst_v):
    pltpu.sync_copy(main2d.at[pl.ds(w * _NMAIN, _NMAIN), :],
                    dst_v.at[pl.ds(0, _NMAIN), :])
    pltpu.sync_copy(extra2d.at[pl.ds(w * _NEXTRA, _NEXTRA), :],
                    dst_v.at[pl.ds(_NMAIN, _NEXTRA), :])


def _deg_body(dst0, dst1, xdst0, xdst1, ones_hbm, zeros_hbm, out,
              didx0, didx1, ones_v, z_v, d0_sh, d1_sh, sem0, sem1):
    c = lax.axis_index("c")
    s = lax.axis_index("s")
    w = c * 16 + s
    pltpu.sync_copy(ones_hbm, ones_v)
    pltpu.sync_copy(zeros_hbm, z_v)
    rowbase = s * _RPS
    for i in range(_RPS // _CHUNK):
        pltpu.sync_copy(z_v, d0_sh.at[pl.ds(rowbase + i * _CHUNK, _CHUNK), :])
        pltpu.sync_copy(z_v, d1_sh.at[pl.ds(rowbase + i * _CHUNK, _CHUNK), :])
    _load_idx(dst0, xdst0, w, didx0)
    _load_idx(dst1, xdst1, w, didx1)
    plsc.subcore_barrier()

    def chunk(k, carry):
        pltpu.async_copy(ones_v, d0_sh.at[didx0.at[k]], sem0, add=True)
        pltpu.async_copy(ones_v, d1_sh.at[didx1.at[k]], sem1, add=True)
        return carry

    lax.fori_loop(0, _NCHUNK, chunk, 0)

    def drain(k, carry):
        pltpu.make_async_copy(ones_v, d0_sh.at[didx0.at[k]], sem0).wait()
        pltpu.make_async_copy(ones_v, d1_sh.at[didx1.at[k]], sem1).wait()
        return carry

    lax.fori_loop(0, _NCHUNK, drain, 0)
    plsc.subcore_barrier()
    rows = pl.ds(rowbase, _RPS)
    pltpu.sync_copy(d0_sh.at[rows, :], out.at[c, 0, rows, :])
    pltpu.sync_copy(d1_sh.at[rows, :], out.at[c, 1, rows, :])


def _deg_call(dst0, dst1, xdst0, xdst1):
    ones = jnp.ones((_CHUNK, 16), jnp.float32)
    zeros = jnp.zeros((_CHUNK, 16), jnp.float32)
    fn = pl.kernel(
        _deg_body,
        out_type=jax.ShapeDtypeStruct((2, 2, _NP, 16), jnp.float32),
        mesh=_mesh(),
        scratch_types=[
            pltpu.VMEM((_NCHUNK, _CHUNK), jnp.int32),
            pltpu.VMEM((_NCHUNK, _CHUNK), jnp.int32),
            pltpu.VMEM((_CHUNK, 16), jnp.float32),
            pltpu.VMEM((_CHUNK, 16), jnp.float32),
            pltpu.VMEM_SHARED((_NP, 16), jnp.float32),
            pltpu.VMEM_SHARED((_NP, 16), jnp.float32),
            pltpu.SemaphoreType.DMA,
            pltpu.SemaphoreType.DMA,
        ],
        compiler_params=_SC_PARAMS,
    )
    return fn(dst0, dst1, xdst0, xdst1, ones, zeros)


_NBUF = 6
_LOOK = 3


def _prop_body(g, src, dst, xsrc, xdst, zeros_hbm, out, sidx, didx, rows_v,
               agg_sh, gsem, ssem):
    c = lax.axis_index("c")
    s = lax.axis_index("s")
    w = c * 16 + s
    z_v = rows_v.at[0]
    pltpu.sync_copy(zeros_hbm, z_v)
    rowbase = s * _RPS
    for i in range(_RPS // _CHUNK):
        pltpu.sync_copy(z_v, agg_sh.at[pl.ds(rowbase + i * _CHUNK, _CHUNK), :])
    _load_idx(src, xsrc, w, sidx)
    _load_idx(dst, xdst, w, didx)
    plsc.subcore_barrier()

    def fire_gather(k):
        slot = lax.rem(k, _NBUF)
        pltpu.async_copy(g.at[sidx.at[k]], rows_v.at[slot], gsem.at[slot])

    def wait_gather(k):
        slot = lax.rem(k, _NBUF)
        pltpu.make_async_copy(g.at[sidx.at[k]], rows_v.at[slot],
                              gsem.at[slot]).wait()

    def fire_scatter(k):
        slot = lax.rem(k, _NBUF)
        pltpu.async_copy(rows_v.at[slot], agg_sh.at[didx.at[k]],
                         ssem.at[slot], add=True)

    def wait_scatter(k):
        slot = lax.rem(k, _NBUF)
        pltpu.make_async_copy(rows_v.at[slot], agg_sh.at[didx.at[k]],
                              ssem.at[slot]).wait()

    for b in range(_LOOK):
        fire_gather(b)

    def warm(k, carry):
        wait_gather(k)
        fire_scatter(k)
        fire_gather(k + _LOOK)
        return carry

    def steady(k, carry):
        wait_gather(k)
        fire_scatter(k)
        wait_scatter(k + _LOOK - _NBUF)
        fire_gather(k + _LOOK)
        return carry

    def tail(k, carry):
        wait_gather(k)
        fire_scatter(k)
        return carry

    lax.fori_loop(0, _NBUF - _LOOK, warm, 0)
    lax.fori_loop(_NBUF - _LOOK, _NCHUNK - _LOOK, steady, 0)
    lax.fori_loop(_NCHUNK - _LOOK, _NCHUNK, tail, 0)

    def drain(k, carry):
        wait_scatter(k)
        return carry

    lax.fori_loop(_NCHUNK - _NBUF, _NCHUNK, drain, 0)
    plsc.subcore_barrier()
    rows = pl.ds(rowbase, _RPS)
    pltpu.sync_copy(agg_sh.at[rows, :], out.at[c, rows, :])


def _prop_call(g, src, dst, xsrc, xdst):
    zeros = jnp.zeros((_CHUNK, _DH), jnp.float32)
    fn = pl.kernel(
        _prop_body,
        out_type=jax.ShapeDtypeStruct((2, _NP, _DH), jnp.float32),
        mesh=_mesh(),
        scratch_types=[
            pltpu.VMEM((_NCHUNK, _CHUNK), jnp.int32),
            pltpu.VMEM((_NCHUNK, _CHUNK), jnp.int32),
            pltpu.VMEM((_NBUF, _CHUNK, _DH), jnp.float32),
            pltpu.VMEM_SHARED((_NP, _DH), jnp.float32),
            pltpu.SemaphoreType.DMA((_NBUF,)),
            pltpu.SemaphoreType.DMA((_NBUF,)),
        ],
        compiler_params=_SC_PARAMS,
    )
    return fn(g, src, dst, xsrc, xdst, zeros)


def _dinv_from(degt_ref):
    d = degt_ref[:, 0:1] + degt_ref[:, 1:2] + 1.0
    return lax.rsqrt(jnp.maximum(d, 1.0))


def _premlp_body(x_ref, w1_ref, b1_ref, w2_ref, b2_ref, degt_ref,
                 h_ref, dinv_ref, g_ref):
    x = x_ref[...]
    h = jnp.maximum(
        lax.dot_general(x, w1_ref[...], (((1,), (1,)), ((), ())),
                        preferred_element_type=jnp.float32,
                        precision=lax.Precision.HIGHEST) + b1_ref[...], 0.0)
    h = jnp.maximum(
        lax.dot_general(h, w2_ref[...], (((1,), (1,)), ((), ())),
                        preferred_element_type=jnp.float32,
                        precision=lax.Precision.HIGHEST) + b2_ref[...], 0.0)
    dinv = _dinv_from(degt_ref)
    h_ref[...] = h
    dinv_ref[...] = dinv
    g_ref[...] = h * dinv


def _premlp_call(x, w1, b1, w2, b2, degt0):
    n, d_in = x.shape
    return pl.pallas_call(
        _premlp_body,
        grid=(_GRID,),
        in_specs=[
            pl.BlockSpec((_BLK, d_in), lambda i: (i, 0)),
            pl.BlockSpec(w1.shape, lambda i: (0, 0)),
            pl.BlockSpec((1, _DH), lambda i: (0, 0)),
            pl.BlockSpec(w2.shape, lambda i: (0, 0)),
            pl.BlockSpec((1, _DH), lambda i: (0, 0)),
            pl.BlockSpec((_BLK, 2), lambda i: (i, 0)),
        ],
        out_specs=[
            pl.BlockSpec((_BLK, _DH), lambda i: (i, 0)),
            pl.BlockSpec((_BLK, 1), lambda i: (i, 0)),
            pl.BlockSpec((_BLK, _DH), lambda i: (i, 0)),
        ],
        out_shape=[
            jax.ShapeDtypeStruct((n, _DH), jnp.float32),
            jax.ShapeDtypeStruct((n, 1), jnp.float32),
            jax.ShapeDtypeStruct((n, _DH), jnp.float32),
        ],
    )(x, w1, b1, w2, b2, degt0)


def _fuse_body(f_ref, gp_ref, p_ref, dinv_ref, fo_ref, go_ref):
    agg = p_ref[0] + p_ref[1] + gp_ref[...]
    dinv = dinv_ref[...]
    f = f_ref[...] - agg * dinv
    fo_ref[...] = f
    go_ref[...] = f * dinv


def _fuse_call(f, gp, p, dinv):
    return pl.pallas_call(
        _fuse_body,
        grid=(_GRID,),
        in_specs=[
            pl.BlockSpec((_BLK, _DH), lambda i: (i, 0)),
            pl.BlockSpec((_BLK, _DH), lambda i: (i, 0)),
            pl.BlockSpec((2, _BLK, _DH), lambda i: (0, i, 0)),
            pl.BlockSpec((_BLK, 1), lambda i: (i, 0)),
        ],
        out_specs=[
            pl.BlockSpec((_BLK, _DH), lambda i: (i, 0)),
            pl.BlockSpec((_BLK, _DH), lambda i: (i, 0)),
        ],
        out_shape=[
            jax.ShapeDtypeStruct((_N, _DH), jnp.float32),
            jax.ShapeDtypeStruct((_N, _DH), jnp.float32),
        ],
    )(f, gp, p, dinv)


def _poly_out(f0, f1, f2, w3, b3):
    acc = jnp.broadcast_to(b3, (f0.shape[0], _DH))
    fs = (f0, f1, f2)
    for k in range(3):
        m_k = None
        for j in range(3):
            t = _THETA[j][k]
            if t == 0.0:
                continue
            blk = w3[:, _DH * j:_DH * (j + 1)] * t
            m_k = blk if m_k is None else m_k + blk
        acc = acc + lax.dot_general(fs[k], m_k, (((1,), (1,)), ((), ())),
                                    preferred_element_type=jnp.float32,
                                    precision=lax.Precision.HIGHEST)
    return acc


def _mid_body(f0_ref, f1_ref, g1_ref, q_ref, dinv_ref, w3_ref, b3_ref,
              degt_ref, h_ref, dinv1_ref, g_ref):
    dinv = dinv_ref[...]
    f2 = f1_ref[...] - (q_ref[0] + q_ref[1] + g1_ref[...]) * dinv
    h = _poly_out(f0_ref[...], f1_ref[...], f2, w3_ref[...], b3_ref[...])
    dinv1 = _dinv_from(degt_ref)
    h_ref[...] = h
    dinv1_ref[...] = dinv1
    g_ref[...] = h * dinv1


def _mid_call(f0, f1, g1, q, dinv, w3, b3, degt1):
    return pl.pallas_call(
        _mid_body,
        grid=(_GRID,),
        in_specs=[
            pl.BlockSpec((_BLK, _DH), lambda i: (i, 0)),
            pl.BlockSpec((_BLK, _DH), lambda i: (i, 0)),
            pl.BlockSpec((_BLK, _DH), lambda i: (i, 0)),
            pl.BlockSpec((2, _BLK, _DH), lambda i: (0, i, 0)),
            pl.BlockSpec((_BLK, 1), lambda i: (i, 0)),
            pl.BlockSpec(w3.shape, lambda i: (0, 0)),
            pl.BlockSpec((1, _DH), lambda i: (0, 0)),
            pl.BlockSpec((_BLK, 2), lambda i: (i, 0)),
        ],
        out_specs=[
            pl.BlockSpec((_BLK, _DH), lambda i: (i, 0)),
            pl.BlockSpec((_BLK, 1), lambda i: (i, 0)),
            pl.BlockSpec((_BLK, _DH), lambda i: (i, 0)),
        ],
        out_shape=[
            jax.ShapeDtypeStruct((_N, _DH), jnp.float32),
            jax.ShapeDtypeStruct((_N, 1), jnp.float32),
            jax.ShapeDtypeStruct((_N, _DH), jnp.float32),
        ],
    )(f0, f1, g1, q, dinv, w3, b3, degt1)


def _final_body(f0_ref, f1_ref, g1_ref, q_ref, dinv_ref, w3_ref, b3_ref,
                w4_ref, b4_ref, out_ref):
    dinv = dinv_ref[...]
    f2 = f1_ref[...] - (q_ref[0] + q_ref[1] + g1_ref[...]) * dinv
    f0 = f0_ref[...]
    h2 = _poly_out(f0, f1_ref[...], f2, w3_ref[...], b3_ref[...])
    hs = jnp.maximum(f0 + h2, 0.0)
    out_ref[...] = lax.dot_general(
        hs, w4_ref[...], (((1,), (1,)), ((), ())),
        preferred_element_type=jnp.float32,
        precision=lax.Precision.HIGHEST) + b4_ref[...]


def _final_call(f0, f1, g1, q, dinv, w3, b3, w4, b4):
    n_cls = w4.shape[0]
    return pl.pallas_call(
        _final_body,
        grid=(_GRID,),
        in_specs=[
            pl.BlockSpec((_BLK, _DH), lambda i: (i, 0)),
            pl.BlockSpec((_BLK, _DH), lambda i: (i, 0)),
            pl.BlockSpec((_BLK, _DH), lambda i: (i, 0)),
            pl.BlockSpec((2, _BLK, _DH), lambda i: (0, i, 0)),
            pl.BlockSpec((_BLK, 1), lambda i: (i, 0)),
            pl.BlockSpec(w3.shape, lambda i: (0, 0)),
            pl.BlockSpec((1, _DH), lambda i: (0, 0)),
            pl.BlockSpec(w4.shape, lambda i: (0, 0)),
            pl.BlockSpec((1, n_cls), lambda i: (0, 0)),
        ],
        out_specs=pl.BlockSpec((_BLK, n_cls), lambda i: (i, 0)),
        out_shape=jax.ShapeDtypeStruct((_N, n_cls), jnp.float32),
    )(f0, f1, g1, q, dinv, w3, b3, w4, b4)


def _split_edges(ei):
    main = _NMAIN * 32 * _CHUNK
    src2d = ei[0].reshape(_E // _CHUNK, _CHUNK)
    dst2d = ei[1].reshape(_E // _CHUNK, _CHUNK)
    xsrc = jnp.concatenate([ei[0, main:], _PAD_SRC]).reshape(64, _CHUNK)
    xdst = jnp.concatenate([ei[1, main:], _PAD_DST]).reshape(64, _CHUNK)
    return src2d, dst2d, xsrc, xdst


def kernel(in_feat, edge_index_r0, edge_index_r1, W1, b1, W2, b2, W3, b3,
           W4, b4, relation_weights):
    del relation_weights
    src0, dst0, xsrc0, xdst0 = _split_edges(edge_index_r0)
    src1, dst1, xsrc1, xdst1 = _split_edges(edge_index_r1)
    b1r = b1.reshape(1, -1)
    b2r = b2.reshape(1, -1)
    b3r = b3.reshape(1, -1)
    b4r = b4.reshape(1, -1)

    degp = _deg_call(dst0, dst1, xdst0, xdst1)
    degt0 = degp[:, 0, :_N, 0].T
    degt1 = degp[:, 1, :_N, 0].T

    h, dinv, g0 = _premlp_call(in_feat, W1, b1r, W2, b2r, degt0)
    p = _prop_call(g0, src0, dst0, xsrc0, xdst0)
    f1, g1 = _fuse_call(h, g0, p, dinv)
    q = _prop_call(g1, src0, dst0, xsrc0, xdst0)
    h1, dinv1, g0b = _mid_call(h, f1, g1, q, dinv, W3, b3r, degt1)

    p = _prop_call(g0b, src1, dst1, xsrc1, xdst1)
    f1b, g1b = _fuse_call(h1, g0b, p, dinv1)
    q = _prop_call(g1b, src1, dst1, xsrc1, xdst1)
    return _final_call(h1, f1b, g1b, q, dinv1, W3, b3r, W4, b4r)


# speedup vs baseline: 1.1431x; 1.1431x over previous
import jax
import jax.numpy as jnp
import numpy as np
from jax import lax
from jax.experimental import pallas as pl
from jax.experimental.pallas import tpu as pltpu
from jax.experimental.pallas import tpu_sc as plsc

_THETA = ((3.0, -3.0, 0.75), (0.0, 3.0, -1.5), (0.0, 0.0, 0.75))

_N = 10000
_NP = 10240
_E = 320000
_CHUNK = 128
_NCHUNK = 80
_NMAIN = 78
_NEXTRA = 2
_RPS = _NP // 16
_DH = 64

_BLK = 2000
_GRID = _N // _BLK

_NPAD_E = 64 * _CHUNK - 512
_PAD_SRC = np.asarray((np.arange(_NPAD_E) * 97) % _N, np.int32)
_PAD_DST = np.asarray(_N + np.arange(_NPAD_E) % (_NP - _N), np.int32)


def _mesh():
    return plsc.VectorSubcoreMesh(core_axis_name="c", subcore_axis_name="s")


_SC_PARAMS = pltpu.CompilerParams(use_tc_tiling_on_sc=False)


def _load_idx(main2d, extra2d, w, dst_v):
    pltpu.sync_copy(main2d.at[pl.ds(w * _NMAIN, _NMAIN), :],
                    dst_v.at[pl.ds(0, _NMAIN), :])
    pltpu.sync_copy(extra2d.at[pl.ds(w * _NEXTRA, _NEXTRA), :],
                    dst_v.at[pl.ds(_NMAIN, _NEXTRA), :])


def _deg_body(dst0, dst1, xdst0, xdst1, ones_hbm, zeros_hbm, out,
              didx0, didx1, ones_v, z_v, d0_sh, d1_sh, sem0, sem1):
    c = lax.axis_index("c")
    s = lax.axis_index("s")
    w = c * 16 + s
    pltpu.sync_copy(ones_hbm, ones_v)
    pltpu.sync_copy(zeros_hbm, z_v)
    rowbase = s * _RPS
    for i in range(_RPS // _CHUNK):
        pltpu.sync_copy(z_v, d0_sh.at[pl.ds(rowbase + i * _CHUNK, _CHUNK), :])
        pltpu.sync_copy(z_v, d1_sh.at[pl.ds(rowbase + i * _CHUNK, _CHUNK), :])
    _load_idx(dst0, xdst0, w, didx0)
    _load_idx(dst1, xdst1, w, didx1)
    plsc.subcore_barrier()

    def chunk(k, carry):
        pltpu.async_copy(ones_v, d0_sh.at[didx0.at[k]], sem0, add=True)
        pltpu.async_copy(ones_v, d1_sh.at[didx1.at[k]], sem1, add=True)
        return carry

    lax.fori_loop(0, _NCHUNK, chunk, 0)

    def drain(k, carry):
        pltpu.make_async_copy(ones_v, d0_sh.at[didx0.at[k]], sem0).wait()
        pltpu.make_async_copy(ones_v, d1_sh.at[didx1.at[k]], sem1).wait()
        return carry

    lax.fori_loop(0, _NCHUNK, drain, 0)
    plsc.subcore_barrier()
    rows = pl.ds(rowbase, _RPS)
    pltpu.sync_copy(d0_sh.at[rows, :], out.at[c, 0, rows, :])
    pltpu.sync_copy(d1_sh.at[rows, :], out.at[c, 1, rows, :])


def _deg_call(dst0, dst1, xdst0, xdst1):
    ones = jnp.ones((_CHUNK, 16), jnp.float32)
    zeros = jnp.zeros((_CHUNK, 16), jnp.float32)
    fn = pl.kernel(
        _deg_body,
        out_type=jax.ShapeDtypeStruct((2, 2, _NP, 16), jnp.float32),
        mesh=_mesh(),
        scratch_types=[
            pltpu.VMEM((_NCHUNK, _CHUNK), jnp.int32),
            pltpu.VMEM((_NCHUNK, _CHUNK), jnp.int32),
            pltpu.VMEM((_CHUNK, 16), jnp.float32),
            pltpu.VMEM((_CHUNK, 16), jnp.float32),
            pltpu.VMEM_SHARED((_NP, 16), jnp.float32),
            pltpu.VMEM_SHARED((_NP, 16), jnp.float32),
            pltpu.SemaphoreType.DMA,
            pltpu.SemaphoreType.DMA,
        ],
        compiler_params=_SC_PARAMS,
    )
    return fn(dst0, dst1, xdst0, xdst1, ones, zeros)


_NBUF = 6
_LOOK = 3


def _prop_body(g, src, dst, xsrc, xdst, zeros_hbm, out, sidx, didx, rows_v,
               agg_sh, gsem, ssem):
    c = lax.axis_index("c")
    s = lax.axis_index("s")
    w = c * 16 + s
    z_v = rows_v.at[0]
    pltpu.sync_copy(zeros_hbm, z_v)
    rowbase = s * _RPS
    for i in range(_RPS // _CHUNK):
        pltpu.sync_copy(z_v, agg_sh.at[pl.ds(rowbase + i * _CHUNK, _CHUNK), :])
    _load_idx(src, xsrc, w, sidx)
    _load_idx(dst, xdst, w, didx)
    plsc.subcore_barrier()

    def fire_gather(k):
        slot = lax.rem(k, _NBUF)
        pltpu.async_copy(g.at[sidx.at[k]], rows_v.at[slot], gsem.at[slot])

    def wait_gather(k):
        slot = lax.rem(k, _NBUF)
        pltpu.make_async_copy(g.at[sidx.at[k]], rows_v.at[slot],
                              gsem.at[slot]).wait()

    def fire_scatter(k):
        slot = lax.rem(k, _NBUF)
        pltpu.async_copy(rows_v.at[slot], agg_sh.at[didx.at[k]],
                         ssem.at[slot], add=True)

    def wait_scatter(k):
        slot = lax.rem(k, _NBUF)
        pltpu.make_async_copy(rows_v.at[slot], agg_sh.at[didx.at[k]],
                              ssem.at[slot]).wait()

    for b in range(_LOOK):
        fire_gather(b)

    def warm(k, carry):
        wait_gather(k)
        fire_scatter(k)
        fire_gather(k + _LOOK)
        return carry

    def steady(k, carry):
        wait_gather(k)
        fire_scatter(k)
        wait_scatter(k + _LOOK - _NBUF)
        fire_gather(k + _LOOK)
        return carry

    def tail(k, carry):
        wait_gather(k)
        fire_scatter(k)
        return carry

    lax.fori_loop(0, _NBUF - _LOOK, warm, 0)
    lax.fori_loop(_NBUF - _LOOK, _NCHUNK - _LOOK, steady, 0)
    lax.fori_loop(_NCHUNK - _LOOK, _NCHUNK, tail, 0)

    def drain(k, carry):
        wait_scatter(k)
        return carry

    lax.fori_loop(_NCHUNK - _NBUF, _NCHUNK, drain, 0)
    plsc.subcore_barrier()
    rows = pl.ds(rowbase, _RPS)
    pltpu.sync_copy(agg_sh.at[rows, :], out.at[c, rows, :])


def _prop_call(g, src, dst, xsrc, xdst):
    zeros = jnp.zeros((_CHUNK, _DH), jnp.float32)
    fn = pl.kernel(
        _prop_body,
        out_type=jax.ShapeDtypeStruct((2, _NP, _DH), jnp.float32),
        mesh=_mesh(),
        scratch_types=[
            pltpu.VMEM((_NCHUNK, _CHUNK), jnp.int32),
            pltpu.VMEM((_NCHUNK, _CHUNK), jnp.int32),
            pltpu.VMEM((_NBUF, _CHUNK, _DH), jnp.float32),
            pltpu.VMEM_SHARED((_NP, _DH), jnp.float32),
            pltpu.SemaphoreType.DMA((_NBUF,)),
            pltpu.SemaphoreType.DMA((_NBUF,)),
        ],
        compiler_params=_SC_PARAMS,
    )
    return fn(g, src, dst, xsrc, xdst, zeros)


def _dinv_from(degp_ref):
    d = degp_ref[0, 0, :, 0:1] + degp_ref[1, 0, :, 0:1] + 1.0
    return lax.rsqrt(jnp.maximum(d, 1.0))


def _premlp_body(x_ref, w1_ref, b1_ref, w2_ref, b2_ref, degp_ref,
                 h_ref, dinv_ref, g_ref):
    x = x_ref[...]
    h = jnp.maximum(
        lax.dot_general(x, w1_ref[...], (((1,), (1,)), ((), ())),
                        preferred_element_type=jnp.float32,
                        precision=lax.Precision.HIGHEST) + b1_ref[...], 0.0)
    h = jnp.maximum(
        lax.dot_general(h, w2_ref[...], (((1,), (1,)), ((), ())),
                        preferred_element_type=jnp.float32,
                        precision=lax.Precision.HIGHEST) + b2_ref[...], 0.0)
    dinv = _dinv_from(degp_ref)
    h_ref[...] = h
    dinv_ref[...] = dinv
    g_ref[...] = h * dinv


def _premlp_call(x, w1, b1, w2, b2, degp):
    n, d_in = x.shape
    return pl.pallas_call(
        _premlp_body,
        grid=(_GRID,),
        in_specs=[
            pl.BlockSpec((_BLK, d_in), lambda i: (i, 0)),
            pl.BlockSpec(w1.shape, lambda i: (0, 0)),
            pl.BlockSpec((1, _DH), lambda i: (0, 0)),
            pl.BlockSpec(w2.shape, lambda i: (0, 0)),
            pl.BlockSpec((1, _DH), lambda i: (0, 0)),
            pl.BlockSpec((2, 1, _BLK, 16), lambda i: (0, 0, i, 0)),
        ],
        out_specs=[
            pl.BlockSpec((_BLK, _DH), lambda i: (i, 0)),
            pl.BlockSpec((_BLK, 1), lambda i: (i, 0)),
            pl.BlockSpec((_BLK, _DH), lambda i: (i, 0)),
        ],
        out_shape=[
            jax.ShapeDtypeStruct((n, _DH), jnp.float32),
            jax.ShapeDtypeStruct((n, 1), jnp.float32),
            jax.ShapeDtypeStruct((n, _DH), jnp.float32),
        ],
    )(x, w1, b1, w2, b2, degp)


def _fuse_body(f_ref, gp_ref, p_ref, dinv_ref, fo_ref, go_ref):
    agg = p_ref[0] + p_ref[1] + gp_ref[...]
    dinv = dinv_ref[...]
    f = f_ref[...] - agg * dinv
    fo_ref[...] = f
    go_ref[...] = f * dinv


def _fuse_call(f, gp, p, dinv):
    return pl.pallas_call(
        _fuse_body,
        grid=(_GRID,),
        in_specs=[
            pl.BlockSpec((_BLK, _DH), lambda i: (i, 0)),
            pl.BlockSpec((_BLK, _DH), lambda i: (i, 0)),
            pl.BlockSpec((2, _BLK, _DH), lambda i: (0, i, 0)),
            pl.BlockSpec((_BLK, 1), lambda i: (i, 0)),
        ],
        out_specs=[
            pl.BlockSpec((_BLK, _DH), lambda i: (i, 0)),
            pl.BlockSpec((_BLK, _DH), lambda i: (i, 0)),
        ],
        out_shape=[
            jax.ShapeDtypeStruct((_N, _DH), jnp.float32),
            jax.ShapeDtypeStruct((_N, _DH), jnp.float32),
        ],
    )(f, gp, p, dinv)


def _poly_out(f0, f1, f2, w3, b3):
    acc = jnp.broadcast_to(b3, (f0.shape[0], _DH))
    fs = (f0, f1, f2)
    for k in range(3):
        m_k = None
        for j in range(3):
            t = _THETA[j][k]
            if t == 0.0:
                continue
            blk = w3[:, _DH * j:_DH * (j + 1)] * t
            m_k = blk if m_k is None else m_k + blk
        acc = acc + lax.dot_general(fs[k], m_k, (((1,), (1,)), ((), ())),
                                    preferred_element_type=jnp.float32,
                                    precision=lax.Precision.HIGHEST)
    return acc


def _mid_body(f0_ref, f1_ref, g1_ref, q_ref, dinv_ref, w3_ref, b3_ref,
              degp_ref, h_ref, dinv1_ref, g_ref):
    dinv = dinv_ref[...]
    f2 = f1_ref[...] - (q_ref[0] + q_ref[1] + g1_ref[...]) * dinv
    h = _poly_out(f0_ref[...], f1_ref[...], f2, w3_ref[...], b3_ref[...])
    dinv1 = _dinv_from(degp_ref)
    h_ref[...] = h
    dinv1_ref[...] = dinv1
    g_ref[...] = h * dinv1


def _mid_call(f0, f1, g1, q, dinv, w3, b3, degp):
    return pl.pallas_call(
        _mid_body,
        grid=(_GRID,),
        in_specs=[
            pl.BlockSpec((_BLK, _DH), lambda i: (i, 0)),
            pl.BlockSpec((_BLK, _DH), lambda i: (i, 0)),
            pl.BlockSpec((_BLK, _DH), lambda i: (i, 0)),
            pl.BlockSpec((2, _BLK, _DH), lambda i: (0, i, 0)),
            pl.BlockSpec((_BLK, 1), lambda i: (i, 0)),
            pl.BlockSpec(w3.shape, lambda i: (0, 0)),
            pl.BlockSpec((1, _DH), lambda i: (0, 0)),
            pl.BlockSpec((2, 1, _BLK, 16), lambda i: (0, 1, i, 0)),
        ],
        out_specs=[
            pl.BlockSpec((_BLK, _DH), lambda i: (i, 0)),
            pl.BlockSpec((_BLK, 1), lambda i: (i, 0)),
            pl.BlockSpec((_BLK, _DH), lambda i: (i, 0)),
        ],
        out_shape=[
            jax.ShapeDtypeStruct((_N, _DH), jnp.float32),
            jax.ShapeDtypeStruct((_N, 1), jnp.float32),
            jax.ShapeDtypeStruct((_N, _DH), jnp.float32),
        ],
    )(f0, f1, g1, q, dinv, w3, b3, degp)


def _final_body(f0_ref, f1_ref, g1_ref, q_ref, dinv_ref, w3_ref, b3_ref,
                w4_ref, b4_ref, out_ref):
    dinv = dinv_ref[...]
    f2 = f1_ref[...] - (q_ref[0] + q_ref[1] + g1_ref[...]) * dinv
    f0 = f0_ref[...]
    h2 = _poly_out(f0, f1_ref[...], f2, w3_ref[...], b3_ref[...])
    hs = jnp.maximum(f0 + h2, 0.0)
    out_ref[...] = lax.dot_general(
        hs, w4_ref[...], (((1,), (1,)), ((), ())),
        preferred_element_type=jnp.float32,
        precision=lax.Precision.HIGHEST) + b4_ref[...]


def _final_call(f0, f1, g1, q, dinv, w3, b3, w4, b4):
    n_cls = w4.shape[0]
    return pl.pallas_call(
        _final_body,
        grid=(_GRID,),
        in_specs=[
            pl.BlockSpec((_BLK, _DH), lambda i: (i, 0)),
            pl.BlockSpec((_BLK, _DH), lambda i: (i, 0)),
            pl.BlockSpec((_BLK, _DH), lambda i: (i, 0)),
            pl.BlockSpec((2, _BLK, _DH), lambda i: (0, i, 0)),
            pl.BlockSpec((_BLK, 1), lambda i: (i, 0)),
            pl.BlockSpec(w3.shape, lambda i: (0, 0)),
            pl.BlockSpec((1, _DH), lambda i: (0, 0)),
            pl.BlockSpec(w4.shape, lambda i: (0, 0)),
            pl.BlockSpec((1, n_cls), lambda i: (0, 0)),
        ],
        out_specs=pl.BlockSpec((_BLK, n_cls), lambda i: (i, 0)),
        out_shape=jax.ShapeDtypeStruct((_N, n_cls), jnp.float32),
    )(f0, f1, g1, q, dinv, w3, b3, w4, b4)


def _split_edges(ei):
    main = _NMAIN * 32 * _CHUNK
    src2d = ei[0].reshape(_E // _CHUNK, _CHUNK)
    dst2d = ei[1].reshape(_E // _CHUNK, _CHUNK)
    xsrc = jnp.concatenate([ei[0, main:], _PAD_SRC]).reshape(64, _CHUNK)
    xdst = jnp.concatenate([ei[1, main:], _PAD_DST]).reshape(64, _CHUNK)
    return src2d, dst2d, xsrc, xdst


def kernel(in_feat, edge_index_r0, edge_index_r1, W1, b1, W2, b2, W3, b3,
           W4, b4, relation_weights):
    del relation_weights
    src0, dst0, xsrc0, xdst0 = _split_edges(edge_index_r0)
    src1, dst1, xsrc1, xdst1 = _split_edges(edge_index_r1)
    b1r = b1.reshape(1, -1)
    b2r = b2.reshape(1, -1)
    b3r = b3.reshape(1, -1)
    b4r = b4.reshape(1, -1)

    degp = _deg_call(dst0, dst1, xdst0, xdst1)

    h, dinv, g0 = _premlp_call(in_feat, W1, b1r, W2, b2r, degp)
    p = _prop_call(g0, src0, dst0, xsrc0, xdst0)
    f1, g1 = _fuse_call(h, g0, p, dinv)
    q = _prop_call(g1, src0, dst0, xsrc0, xdst0)
    h1, dinv1, g0b = _mid_call(h, f1, g1, q, dinv, W3, b3r, degp)

    p = _prop_call(g0b, src1, dst1, xsrc1, xdst1)
    f1b, g1b = _fuse_call(h1, g0b, p, dinv1)
    q = _prop_call(g1b, src1, dst1, xsrc1, xdst1)
    return _final_call(h1, f1b, g1b, q, dinv1, W3, b3r, W4, b4r)


# speedup vs baseline: 1.1880x; 1.0393x over previous
import functools

import jax
import jax.numpy as jnp
from jax import lax
from jax.experimental import pallas as pl
from jax.experimental.pallas import tpu as pltpu
from jax.experimental.pallas import tpu_sc as plsc

_THETA = ((3.0, -3.0, 0.75), (0.0, 3.0, -1.5), (0.0, 0.0, 0.75))

_N = 10000
_NP = 10240
_E = 320000
_EP = 327680
_CHUNK = 128
_NCHUNK = _EP // (32 * _CHUNK)
_RPS = _NP // 16
_DH = 64

_BLK = 2000
_GRID = _N // _BLK


def _mesh():
    return plsc.VectorSubcoreMesh(core_axis_name="c", subcore_axis_name="s")


_SC_PARAMS = pltpu.CompilerParams(use_tc_tiling_on_sc=False)


def _deg_body(dst0, dst1, ones_hbm, zeros_hbm, out, didx0, didx1, ones_v,
              z_v, d0_sh, d1_sh, sem0, sem1):
    c = lax.axis_index("c")
    s = lax.axis_index("s")
    w = c * 16 + s
    pltpu.sync_copy(ones_hbm, ones_v)
    pltpu.sync_copy(zeros_hbm, z_v)
    rowbase = s * _RPS
    for i in range(_RPS // _CHUNK):
        pltpu.sync_copy(z_v, d0_sh.at[pl.ds(rowbase + i * _CHUNK, _CHUNK), :])
        pltpu.sync_copy(z_v, d1_sh.at[pl.ds(rowbase + i * _CHUNK, _CHUNK), :])
    pltpu.sync_copy(dst0.at[pl.ds(w * _NCHUNK, _NCHUNK), :], didx0)
    pltpu.sync_copy(dst1.at[pl.ds(w * _NCHUNK, _NCHUNK), :], didx1)
    plsc.subcore_barrier()

    def chunk(k, carry):
        pltpu.async_copy(ones_v, d0_sh.at[didx0.at[k]], sem0, add=True)
        pltpu.async_copy(ones_v, d1_sh.at[didx1.at[k]], sem1, add=True)
        return carry

    lax.fori_loop(0, _NCHUNK, chunk, 0)

    def drain(k, carry):
        pltpu.make_async_copy(ones_v, d0_sh.at[didx0.at[k]], sem0).wait()
        pltpu.make_async_copy(ones_v, d1_sh.at[didx1.at[k]], sem1).wait()
        return carry

    lax.fori_loop(0, _NCHUNK, drain, 0)
    plsc.subcore_barrier()
    rows = pl.ds(rowbase, _RPS)
    pltpu.sync_copy(d0_sh.at[rows, :], out.at[c, 0, rows, :])
    pltpu.sync_copy(d1_sh.at[rows, :], out.at[c, 1, rows, :])


def _deg_call(dst0, dst1):
    ones = jnp.ones((_CHUNK, 16), jnp.float32)
    zeros = jnp.zeros((_CHUNK, 16), jnp.float32)
    fn = pl.kernel(
        _deg_body,
        out_type=jax.ShapeDtypeStruct((2, 2, _NP, 16), jnp.float32),
        mesh=_mesh(),
        scratch_types=[
            pltpu.VMEM((_NCHUNK, _CHUNK), jnp.int32),
            pltpu.VMEM((_NCHUNK, _CHUNK), jnp.int32),
            pltpu.VMEM((_CHUNK, 16), jnp.float32),
            pltpu.VMEM((_CHUNK, 16), jnp.float32),
            pltpu.VMEM_SHARED((_NP, 16), jnp.float32),
            pltpu.VMEM_SHARED((_NP, 16), jnp.float32),
            pltpu.SemaphoreType.DMA,
            pltpu.SemaphoreType.DMA,
        ],
        compiler_params=_SC_PARAMS,
    )
    return fn(dst0, dst1, ones, zeros)


_NBUF = 7
_LOOK = 4


def _prop_body(g, src, dst, zeros_hbm, out, sidx, didx, rows_v, agg_sh,
               gsem, ssem):
    c = lax.axis_index("c")
    s = lax.axis_index("s")
    w = c * 16 + s
    z_v = rows_v.at[0]
    pltpu.sync_copy(zeros_hbm, z_v)
    rowbase = s * _RPS
    for i in range(_RPS // _CHUNK):
        pltpu.sync_copy(z_v, agg_sh.at[pl.ds(rowbase + i * _CHUNK, _CHUNK), :])
    pltpu.sync_copy(src.at[pl.ds(w * _NCHUNK, _NCHUNK), :], sidx)
    pltpu.sync_copy(dst.at[pl.ds(w * _NCHUNK, _NCHUNK), :], didx)
    plsc.subcore_barrier()

    def fire_gather(k):
        slot = lax.rem(k, _NBUF)
        pltpu.async_copy(g.at[sidx.at[k]], rows_v.at[slot], gsem.at[slot])

    def wait_gather(k):
        slot = lax.rem(k, _NBUF)
        pltpu.make_async_copy(g.at[sidx.at[k]], rows_v.at[slot],
                              gsem.at[slot]).wait()

    def fire_scatter(k):
        slot = lax.rem(k, _NBUF)
        pltpu.async_copy(rows_v.at[slot], agg_sh.at[didx.at[k]],
                         ssem.at[slot], add=True)

    def wait_scatter(k):
        slot = lax.rem(k, _NBUF)
        pltpu.make_async_copy(rows_v.at[slot], agg_sh.at[didx.at[k]],
                              ssem.at[slot]).wait()

    for b in range(_LOOK):
        fire_gather(b)

    def warm(k, carry):
        wait_gather(k)
        fire_scatter(k)
        fire_gather(k + _LOOK)
        return carry

    def steady(k, carry):
        wait_gather(k)
        fire_scatter(k)
        wait_scatter(k + _LOOK - _NBUF)
        fire_gather(k + _LOOK)
        return carry

    def tail(k, carry):
        wait_gather(k)
        fire_scatter(k)
        return carry

    lax.fori_loop(0, _NBUF - _LOOK, warm, 0)
    lax.fori_loop(_NBUF - _LOOK, _NCHUNK - _LOOK, steady, 0)
    lax.fori_loop(_NCHUNK - _LOOK, _NCHUNK, tail, 0)

    def drain(k, carry):
        wait_scatter(k)
        return carry

    lax.fori_loop(_NCHUNK - _NBUF, _NCHUNK, drain, 0)
    plsc.subcore_barrier()
    rows = pl.ds(rowbase, _RPS)
    pltpu.sync_copy(agg_sh.at[rows, :], out.at[c, rows, :])


def _prop_call(g, src, dst):
    zeros = jnp.zeros((_CHUNK, _DH), jnp.float32)
    fn = pl.kernel(
        _prop_body,
        out_type=jax.ShapeDtypeStruct((2, _NP, _DH), jnp.float32),
        mesh=_mesh(),
        scratch_types=[
            pltpu.VMEM((_NCHUNK, _CHUNK), jnp.int32),
            pltpu.VMEM((_NCHUNK, _CHUNK), jnp.int32),
            pltpu.VMEM((_NBUF, _CHUNK, _DH), jnp.float32),
            pltpu.VMEM_SHARED((_NP, _DH), jnp.float32),
            pltpu.SemaphoreType.DMA((_NBUF,)),
            pltpu.SemaphoreType.DMA((_NBUF,)),
        ],
        compiler_params=_SC_PARAMS,
    )
    return fn(g, src, dst, zeros)


def _dinv_from(degp_ref):
    d = degp_ref[0, 0, :, 0:1] + degp_ref[1, 0, :, 0:1] + 1.0
    return lax.rsqrt(jnp.maximum(d, 1.0))


def _premlp_body(x_ref, w1_ref, b1_ref, w2_ref, b2_ref, degp_ref,
                 h_ref, dinv_ref, g_ref):
    x = x_ref[...]
    h = jnp.maximum(
        lax.dot_general(x, w1_ref[...], (((1,), (1,)), ((), ())),
                        preferred_element_type=jnp.float32, precision=lax.Precision.HIGHEST) + b1_ref[...], 0.0)
    h = jnp.maximum(
        lax.dot_general(h, w2_ref[...], (((1,), (1,)), ((), ())),
                        preferred_element_type=jnp.float32, precision=lax.Precision.HIGHEST) + b2_ref[...], 0.0)
    dinv = _dinv_from(degp_ref)
    h_ref[...] = h
    dinv_ref[...] = dinv
    g_ref[...] = h * dinv


def _premlp_call(x, w1, b1, w2, b2, degp):
    n, d_in = x.shape
    return pl.pallas_call(
        _premlp_body,
        grid=(_GRID,),
        in_specs=[
            pl.BlockSpec((_BLK, d_in), lambda i: (i, 0)),
            pl.BlockSpec(w1.shape, lambda i: (0, 0)),
            pl.BlockSpec((1, _DH), lambda i: (0, 0)),
            pl.BlockSpec(w2.shape, lambda i: (0, 0)),
            pl.BlockSpec((1, _DH), lambda i: (0, 0)),
            pl.BlockSpec((2, 1, _BLK, 16), lambda i: (0, 0, i, 0)),
        ],
        out_specs=[
            pl.BlockSpec((_BLK, _DH), lambda i: (i, 0)),
            pl.BlockSpec((_BLK, 1), lambda i: (i, 0)),
            pl.BlockSpec((_BLK, _DH), lambda i: (i, 0)),
        ],
        out_shape=[
            jax.ShapeDtypeStruct((n, _DH), jnp.float32),
            jax.ShapeDtypeStruct((n, 1), jnp.float32),
            jax.ShapeDtypeStruct((n, _DH), jnp.float32),
        ],
    )(x, w1, b1, w2, b2, degp)


def _fuse_body(f_ref, gp_ref, p_ref, dinv_ref, fo_ref, go_ref):
    agg = p_ref[0] + p_ref[1] + gp_ref[...]
    dinv = dinv_ref[...]
    f = f_ref[...] - agg * dinv
    fo_ref[...] = f
    go_ref[...] = f * dinv


def _fuse_call(f, gp, p, dinv):
    return pl.pallas_call(
        _fuse_body,
        grid=(_GRID,),
        in_specs=[
            pl.BlockSpec((_BLK, _DH), lambda i: (i, 0)),
            pl.BlockSpec((_BLK, _DH), lambda i: (i, 0)),
            pl.BlockSpec((2, _BLK, _DH), lambda i: (0, i, 0)),
            pl.BlockSpec((_BLK, 1), lambda i: (i, 0)),
        ],
        out_specs=[
            pl.BlockSpec((_BLK, _DH), lambda i: (i, 0)),
            pl.BlockSpec((_BLK, _DH), lambda i: (i, 0)),
        ],
        out_shape=[
            jax.ShapeDtypeStruct((_N, _DH), jnp.float32),
            jax.ShapeDtypeStruct((_N, _DH), jnp.float32),
        ],
    )(f, gp, p, dinv)


def _poly_out(f0, f1, f2, w3, b3):
    acc = jnp.broadcast_to(b3, (f0.shape[0], _DH))
    fs = (f0, f1, f2)
    for k in range(3):
        m_k = None
        for j in range(3):
            t = _THETA[j][k]
            if t == 0.0:
                continue
            blk = w3[:, _DH * j:_DH * (j + 1)] * t
            m_k = blk if m_k is None else m_k + blk
        acc = acc + lax.dot_general(fs[k], m_k, (((1,), (1,)), ((), ())),
                                    preferred_element_type=jnp.float32, precision=lax.Precision.HIGHEST)
    return acc


def _mid_body(f0_ref, f1_ref, g1_ref, q_ref, dinv_ref, w3_ref, b3_ref,
              degp_ref, h_ref, dinv1_ref, g_ref):
    dinv = dinv_ref[...]
    f2 = f1_ref[...] - (q_ref[0] + q_ref[1] + g1_ref[...]) * dinv
    h = _poly_out(f0_ref[...], f1_ref[...], f2, w3_ref[...], b3_ref[...])
    d1 = degp_ref[0, 0, :, 0:1] + degp_ref[1, 0, :, 0:1] + 1.0
    dinv1 = lax.rsqrt(jnp.maximum(d1, 1.0))
    h_ref[...] = h
    dinv1_ref[...] = dinv1
    g_ref[...] = h * dinv1


def _mid_call(f0, f1, g1, q, dinv, w3, b3, degp1):
    return pl.pallas_call(
        _mid_body,
        grid=(_GRID,),
        in_specs=[
            pl.BlockSpec((_BLK, _DH), lambda i: (i, 0)),
            pl.BlockSpec((_BLK, _DH), lambda i: (i, 0)),
            pl.BlockSpec((_BLK, _DH), lambda i: (i, 0)),
            pl.BlockSpec((2, _BLK, _DH), lambda i: (0, i, 0)),
            pl.BlockSpec((_BLK, 1), lambda i: (i, 0)),
            pl.BlockSpec(w3.shape, lambda i: (0, 0)),
            pl.BlockSpec((1, _DH), lambda i: (0, 0)),
            pl.BlockSpec((2, 1, _BLK, 16), lambda i: (0, 1, i, 0)),
        ],
        out_specs=[
            pl.BlockSpec((_BLK, _DH), lambda i: (i, 0)),
            pl.BlockSpec((_BLK, 1), lambda i: (i, 0)),
            pl.BlockSpec((_BLK, _DH), lambda i: (i, 0)),
        ],
        out_shape=[
            jax.ShapeDtypeStruct((_N, _DH), jnp.float32),
            jax.ShapeDtypeStruct((_N, 1), jnp.float32),
            jax.ShapeDtypeStruct((_N, _DH), jnp.float32),
        ],
    )(f0, f1, g1, q, dinv, w3, b3, degp1)


def _final_body(f0_ref, f1_ref, g1_ref, q_ref, dinv_ref, w3_ref, b3_ref,
                w4_ref, b4_ref, out_ref):
    dinv = dinv_ref[...]
    f2 = f1_ref[...] - (q_ref[0] + q_ref[1] + g1_ref[...]) * dinv
    f0 = f0_ref[...]
    h2 = _poly_out(f0, f1_ref[...], f2, w3_ref[...], b3_ref[...])
    hs = jnp.maximum(f0 + h2, 0.0)
    out_ref[...] = lax.dot_general(hs, w4_ref[...], (((1,), (1,)), ((), ())),
                                   preferred_element_type=jnp.float32, precision=lax.Precision.HIGHEST) + b4_ref[...]


def _final_call(f0, f1, g1, q, dinv, w3, b3, w4, b4):
    n_cls = w4.shape[0]
    return pl.pallas_call(
        _final_body,
        grid=(_GRID,),
        in_specs=[
            pl.BlockSpec((_BLK, _DH), lambda i: (i, 0)),
            pl.BlockSpec((_BLK, _DH), lambda i: (i, 0)),
            pl.BlockSpec((_BLK, _DH), lambda i: (i, 0)),
            pl.BlockSpec((2, _BLK, _DH), lambda i: (0, i, 0)),
            pl.BlockSpec((_BLK, 1), lambda i: (i, 0)),
            pl.BlockSpec(w3.shape, lambda i: (0, 0)),
            pl.BlockSpec((1, _DH), lambda i: (0, 0)),
            pl.BlockSpec(w4.shape, lambda i: (0, 0)),
            pl.BlockSpec((1, n_cls), lambda i: (0, 0)),
        ],
        out_specs=pl.BlockSpec((_BLK, n_cls), lambda i: (i, 0)),
        out_shape=jax.ShapeDtypeStruct((_N, n_cls), jnp.float32),
    )(f0, f1, g1, q, dinv, w3, b3, w4, b4)


def _pad_edges(ei):
    pad = _EP - ei.shape[1]
    r = jnp.arange(pad, dtype=ei.dtype)
    src = jnp.concatenate([ei[0], (r * 97) % _N])
    dst = jnp.concatenate([ei[1], _N + r % (_NP - _N)])
    return (src.reshape(_EP // _CHUNK, _CHUNK).astype(jnp.int32),
            dst.reshape(_EP // _CHUNK, _CHUNK).astype(jnp.int32))


def kernel(in_feat, edge_index_r0, edge_index_r1, W1, b1, W2, b2, W3, b3,
           W4, b4, relation_weights):
    del relation_weights
    src0, dst0 = _pad_edges(edge_index_r0)
    src1, dst1 = _pad_edges(edge_index_r1)
    b1r = b1.reshape(1, -1)
    b2r = b2.reshape(1, -1)
    b3r = b3.reshape(1, -1)
    b4r = b4.reshape(1, -1)

    degp = _deg_call(dst0, dst1)

    h, dinv, g0 = _premlp_call(in_feat, W1, b1r, W2, b2r, degp)
    p = _prop_call(g0, src0, dst0)
    f1, g1 = _fuse_call(h, g0, p, dinv)
    q = _prop_call(g1, src0, dst0)
    h1, dinv1, g0b = _mid_call(h, f1, g1, q, dinv, W3, b3r, degp)

    p = _prop_call(g0b, src1, dst1)
    f1b, g1b = _fuse_call(h1, g0b, p, dinv1)
    q = _prop_call(g1b, src1, dst1)
    return _final_call(h1, f1b, g1b, q, dinv1, W3, b3r, W4, b4r)


# speedup vs baseline: 1.2147x; 1.0224x over previous
import functools

import jax
import jax.numpy as jnp
from jax import lax
from jax.experimental import pallas as pl
from jax.experimental.pallas import tpu as pltpu
from jax.experimental.pallas import tpu_sc as plsc

_THETA = ((3.0, -3.0, 0.75), (0.0, 3.0, -1.5), (0.0, 0.0, 0.75))

_N = 10000
_NP = 10240
_E = 320000
_EP = 327680
_CHUNK = 128
_NCHUNK = _EP // (32 * _CHUNK)
_RPS = _NP // 16
_DH = 64

_BLK = 2000
_GRID = _N // _BLK


def _mesh():
    return plsc.VectorSubcoreMesh(core_axis_name="c", subcore_axis_name="s")


_SC_PARAMS = pltpu.CompilerParams(use_tc_tiling_on_sc=False)


def _deg_body(dst0, dst1, ones_hbm, zeros_hbm, out, didx0, didx1, ones_v,
              z_v, d0_sh, d1_sh, sem0, sem1):
    c = lax.axis_index("c")
    s = lax.axis_index("s")
    w = c * 16 + s
    pltpu.sync_copy(ones_hbm, ones_v)
    pltpu.sync_copy(zeros_hbm, z_v)
    rowbase = s * _RPS
    for i in range(_RPS // _CHUNK):
        pltpu.sync_copy(z_v, d0_sh.at[pl.ds(rowbase + i * _CHUNK, _CHUNK), :])
        pltpu.sync_copy(z_v, d1_sh.at[pl.ds(rowbase + i * _CHUNK, _CHUNK), :])
    pltpu.sync_copy(dst0.at[pl.ds(w * _NCHUNK, _NCHUNK), :], didx0)
    pltpu.sync_copy(dst1.at[pl.ds(w * _NCHUNK, _NCHUNK), :], didx1)
    plsc.subcore_barrier()

    def chunk(k, carry):
        pltpu.async_copy(ones_v, d0_sh.at[didx0.at[k]], sem0, add=True)
        pltpu.async_copy(ones_v, d1_sh.at[didx1.at[k]], sem1, add=True)
        return carry

    lax.fori_loop(0, _NCHUNK, chunk, 0)

    def drain(k, carry):
        pltpu.make_async_copy(ones_v, d0_sh.at[didx0.at[k]], sem0).wait()
        pltpu.make_async_copy(ones_v, d1_sh.at[didx1.at[k]], sem1).wait()
        return carry

    lax.fori_loop(0, _NCHUNK, drain, 0)
    plsc.subcore_barrier()
    rows = pl.ds(rowbase, _RPS)
    pltpu.sync_copy(d0_sh.at[rows, :], out.at[c, 0, rows, :])
    pltpu.sync_copy(d1_sh.at[rows, :], out.at[c, 1, rows, :])


def _deg_call(dst0, dst1):
    ones = jnp.ones((_CHUNK, 16), jnp.float32)
    zeros = jnp.zeros((_CHUNK, 16), jnp.float32)
    fn = pl.kernel(
        _deg_body,
        out_type=jax.ShapeDtypeStruct((2, 2, _NP, 16), jnp.float32),
        mesh=_mesh(),
        scratch_types=[
            pltpu.VMEM((_NCHUNK, _CHUNK), jnp.int32),
            pltpu.VMEM((_NCHUNK, _CHUNK), jnp.int32),
            pltpu.VMEM((_CHUNK, 16), jnp.float32),
            pltpu.VMEM((_CHUNK, 16), jnp.float32),
            pltpu.VMEM_SHARED((_NP, 16), jnp.float32),
            pltpu.VMEM_SHARED((_NP, 16), jnp.float32),
            pltpu.SemaphoreType.DMA,
            pltpu.SemaphoreType.DMA,
        ],
        compiler_params=_SC_PARAMS,
    )
    return fn(dst0, dst1, ones, zeros)


_NBUF = 7
_LOOK = 4


def _prop_body(g, src, dst, zeros_hbm, out, sidx, didx, rows_v, agg_sh,
               gsem, ssem):
    c = lax.axis_index("c")
    s = lax.axis_index("s")
    w = c * 16 + s
    z_v = rows_v.at[0]
    pltpu.sync_copy(zeros_hbm, z_v)
    rowbase = s * _RPS
    for i in range(_RPS // _CHUNK):
        pltpu.sync_copy(z_v, agg_sh.at[pl.ds(rowbase + i * _CHUNK, _CHUNK), :])
    pltpu.sync_copy(src.at[pl.ds(w * _NCHUNK, _NCHUNK), :], sidx)
    pltpu.sync_copy(dst.at[pl.ds(w * _NCHUNK, _NCHUNK), :], didx)
    plsc.subcore_barrier()

    def fire_gather(k):
        slot = lax.rem(k, _NBUF)
        pltpu.async_copy(g.at[sidx.at[k]], rows_v.at[slot], gsem.at[slot])

    def wait_gather(k):
        slot = lax.rem(k, _NBUF)
        pltpu.make_async_copy(g.at[sidx.at[k]], rows_v.at[slot],
                              gsem.at[slot]).wait()

    def fire_scatter(k):
        slot = lax.rem(k, _NBUF)
        pltpu.async_copy(rows_v.at[slot], agg_sh.at[didx.at[k]],
                         ssem.at[slot], add=True)

    def wait_scatter(k):
        slot = lax.rem(k, _NBUF)
        pltpu.make_async_copy(rows_v.at[slot], agg_sh.at[didx.at[k]],
                              ssem.at[slot]).wait()

    for b in range(_LOOK):
        fire_gather(b)

    def warm(k, carry):
        wait_gather(k)
        fire_scatter(k)
        fire_gather(k + _LOOK)
        return carry

    def steady(k, carry):
        wait_gather(k)
        fire_scatter(k)
        wait_scatter(k + _LOOK - _NBUF)
        fire_gather(k + _LOOK)
        return carry

    def tail(k, carry):
        wait_gather(k)
        fire_scatter(k)
        return carry

    lax.fori_loop(0, _NBUF - _LOOK, warm, 0)
    lax.fori_loop(_NBUF - _LOOK, _NCHUNK - _LOOK, steady, 0)
    lax.fori_loop(_NCHUNK - _LOOK, _NCHUNK, tail, 0)

    def drain(k, carry):
        wait_scatter(k)
        return carry

    lax.fori_loop(_NCHUNK - _NBUF, _NCHUNK, drain, 0)
    plsc.subcore_barrier()
    rows = pl.ds(rowbase, _RPS)
    pltpu.sync_copy(agg_sh.at[rows, :], out.at[c, rows, :])


def _prop_call(g, src, dst):
    zeros = jnp.zeros((_CHUNK, _DH), jnp.float32)
    fn = pl.kernel(
        _prop_body,
        out_type=jax.ShapeDtypeStruct((2, _NP, _DH), jnp.float32),
        mesh=_mesh(),
        scratch_types=[
            pltpu.VMEM((_NCHUNK, _CHUNK), jnp.int32),
            pltpu.VMEM((_NCHUNK, _CHUNK), jnp.int32),
            pltpu.VMEM((_NBUF, _CHUNK, _DH), jnp.float32),
            pltpu.VMEM_SHARED((_NP, _DH), jnp.float32),
            pltpu.SemaphoreType.DMA((_NBUF,)),
            pltpu.SemaphoreType.DMA((_NBUF,)),
        ],
        compiler_params=_SC_PARAMS,
    )
    return fn(g, src, dst, zeros)


def _dinv_from(degp_ref):
    d = degp_ref[0, 0, :, 0:1] + degp_ref[1, 0, :, 0:1] + 1.0
    return lax.rsqrt(jnp.maximum(d, 1.0))


def _premlp_body(x_ref, w1_ref, b1_ref, w2_ref, b2_ref, h_ref):
    x = x_ref[...]
    h = jnp.maximum(
        lax.dot_general(x, w1_ref[...], (((1,), (1,)), ((), ())),
                        preferred_element_type=jnp.float32, precision=lax.Precision.HIGHEST) + b1_ref[...], 0.0)
    h = jnp.maximum(
        lax.dot_general(h, w2_ref[...], (((1,), (1,)), ((), ())),
                        preferred_element_type=jnp.float32, precision=lax.Precision.HIGHEST) + b2_ref[...], 0.0)
    h_ref[...] = h


def _premlp_call(x, w1, b1, w2, b2):
    n, d_in = x.shape
    return pl.pallas_call(
        _premlp_body,
        grid=(_GRID,),
        in_specs=[
            pl.BlockSpec((_BLK, d_in), lambda i: (i, 0)),
            pl.BlockSpec(w1.shape, lambda i: (0, 0)),
            pl.BlockSpec((1, _DH), lambda i: (0, 0)),
            pl.BlockSpec(w2.shape, lambda i: (0, 0)),
            pl.BlockSpec((1, _DH), lambda i: (0, 0)),
        ],
        out_specs=pl.BlockSpec((_BLK, _DH), lambda i: (i, 0)),
        out_shape=jax.ShapeDtypeStruct((n, _DH), jnp.float32),
    )(x, w1, b1, w2, b2)


def _norm_body(h_ref, degp_ref, dinv_ref, g_ref):
    dinv = _dinv_from(degp_ref)
    dinv_ref[...] = dinv
    g_ref[...] = h_ref[...] * dinv


def _norm_call(h, degp):
    return pl.pallas_call(
        _norm_body,
        grid=(_GRID,),
        in_specs=[
            pl.BlockSpec((_BLK, _DH), lambda i: (i, 0)),
            pl.BlockSpec((2, 1, _BLK, 16), lambda i: (0, 0, i, 0)),
        ],
        out_specs=[
            pl.BlockSpec((_BLK, 1), lambda i: (i, 0)),
            pl.BlockSpec((_BLK, _DH), lambda i: (i, 0)),
        ],
        out_shape=[
            jax.ShapeDtypeStruct((_N, 1), jnp.float32),
            jax.ShapeDtypeStruct((_N, _DH), jnp.float32),
        ],
    )(h, degp)


def _fuse_body(f_ref, gp_ref, p_ref, dinv_ref, fo_ref, go_ref):
    agg = p_ref[0] + p_ref[1] + gp_ref[...]
    dinv = dinv_ref[...]
    f = f_ref[...] - agg * dinv
    fo_ref[...] = f
    go_ref[...] = f * dinv


def _fuse_call(f, gp, p, dinv):
    return pl.pallas_call(
        _fuse_body,
        grid=(_GRID,),
        in_specs=[
            pl.BlockSpec((_BLK, _DH), lambda i: (i, 0)),
            pl.BlockSpec((_BLK, _DH), lambda i: (i, 0)),
            pl.BlockSpec((2, _BLK, _DH), lambda i: (0, i, 0)),
            pl.BlockSpec((_BLK, 1), lambda i: (i, 0)),
        ],
        out_specs=[
            pl.BlockSpec((_BLK, _DH), lambda i: (i, 0)),
            pl.BlockSpec((_BLK, _DH), lambda i: (i, 0)),
        ],
        out_shape=[
            jax.ShapeDtypeStruct((_N, _DH), jnp.float32),
            jax.ShapeDtypeStruct((_N, _DH), jnp.float32),
        ],
    )(f, gp, p, dinv)


def _poly_out(f0, f1, f2, w3, b3):
    acc = jnp.broadcast_to(b3, (f0.shape[0], _DH))
    fs = (f0, f1, f2)
    for k in range(3):
        m_k = None
        for j in range(3):
            t = _THETA[j][k]
            if t == 0.0:
                continue
            blk = w3[:, _DH * j:_DH * (j + 1)] * t
            m_k = blk if m_k is None else m_k + blk
        acc = acc + lax.dot_general(fs[k], m_k, (((1,), (1,)), ((), ())),
                                    preferred_element_type=jnp.float32, precision=lax.Precision.HIGHEST)
    return acc


def _mid_body(f0_ref, f1_ref, g1_ref, q_ref, dinv_ref, w3_ref, b3_ref,
              degp_ref, h_ref, dinv1_ref, g_ref):
    dinv = dinv_ref[...]
    f2 = f1_ref[...] - (q_ref[0] + q_ref[1] + g1_ref[...]) * dinv
    h = _poly_out(f0_ref[...], f1_ref[...], f2, w3_ref[...], b3_ref[...])
    d1 = degp_ref[0, 0, :, 0:1] + degp_ref[1, 0, :, 0:1] + 1.0
    dinv1 = lax.rsqrt(jnp.maximum(d1, 1.0))
    h_ref[...] = h
    dinv1_ref[...] = dinv1
    g_ref[...] = h * dinv1


def _mid_call(f0, f1, g1, q, dinv, w3, b3, degp1):
    return pl.pallas_call(
        _mid_body,
        grid=(_GRID,),
        in_specs=[
            pl.BlockSpec((_BLK, _DH), lambda i: (i, 0)),
            pl.BlockSpec((_BLK, _DH), lambda i: (i, 0)),
            pl.BlockSpec((_BLK, _DH), lambda i: (i, 0)),
            pl.BlockSpec((2, _BLK, _DH), lambda i: (0, i, 0)),
            pl.BlockSpec((_BLK, 1), lambda i: (i, 0)),
            pl.BlockSpec(w3.shape, lambda i: (0, 0)),
            pl.BlockSpec((1, _DH), lambda i: (0, 0)),
            pl.BlockSpec((2, 1, _BLK, 16), lambda i: (0, 1, i, 0)),
        ],
        out_specs=[
            pl.BlockSpec((_BLK, _DH), lambda i: (i, 0)),
            pl.BlockSpec((_BLK, 1), lambda i: (i, 0)),
            pl.BlockSpec((_BLK, _DH), lambda i: (i, 0)),
        ],
        out_shape=[
            jax.ShapeDtypeStruct((_N, _DH), jnp.float32),
            jax.ShapeDtypeStruct((_N, 1), jnp.float32),
            jax.ShapeDtypeStruct((_N, _DH), jnp.float32),
        ],
    )(f0, f1, g1, q, dinv, w3, b3, degp1)


def _final_body(f0_ref, f1_ref, g1_ref, q_ref, dinv_ref, w3_ref, b3_ref,
                w4_ref, b4_ref, out_ref):
    dinv = dinv_ref[...]
    f2 = f1_ref[...] - (q_ref[0] + q_ref[1] + g1_ref[...]) * dinv
    f0 = f0_ref[...]
    h2 = _poly_out(f0, f1_ref[...], f2, w3_ref[...], b3_ref[...])
    hs = jnp.maximum(f0 + h2, 0.0)
    out_ref[...] = lax.dot_general(hs, w4_ref[...], (((1,), (1,)), ((), ())),
                                   preferred_element_type=jnp.float32, precision=lax.Precision.HIGHEST) + b4_ref[...]


def _final_call(f0, f1, g1, q, dinv, w3, b3, w4, b4):
    n_cls = w4.shape[0]
    return pl.pallas_call(
        _final_body,
        grid=(_GRID,),
        in_specs=[
            pl.BlockSpec((_BLK, _DH), lambda i: (i, 0)),
            pl.BlockSpec((_BLK, _DH), lambda i: (i, 0)),
            pl.BlockSpec((_BLK, _DH), lambda i: (i, 0)),
            pl.BlockSpec((2, _BLK, _DH), lambda i: (0, i, 0)),
            pl.BlockSpec((_BLK, 1), lambda i: (i, 0)),
            pl.BlockSpec(w3.shape, lambda i: (0, 0)),
            pl.BlockSpec((1, _DH), lambda i: (0, 0)),
            pl.BlockSpec(w4.shape, lambda i: (0, 0)),
            pl.BlockSpec((1, n_cls), lambda i: (0, 0)),
        ],
        out_specs=pl.BlockSpec((_BLK, n_cls), lambda i: (i, 0)),
        out_shape=jax.ShapeDtypeStruct((_N, n_cls), jnp.float32),
    )(f0, f1, g1, q, dinv, w3, b3, w4, b4)


def _pad_edges(ei):
    pad = _EP - ei.shape[1]
    r = jnp.arange(pad, dtype=ei.dtype)
    src = jnp.concatenate([ei[0], (r * 97) % _N])
    dst = jnp.concatenate([ei[1], _N + r % (_NP - _N)])
    return (src.reshape(_EP // _CHUNK, _CHUNK).astype(jnp.int32),
            dst.reshape(_EP // _CHUNK, _CHUNK).astype(jnp.int32))


def kernel(in_feat, edge_index_r0, edge_index_r1, W1, b1, W2, b2, W3, b3,
           W4, b4, relation_weights):
    del relation_weights
    src0, dst0 = _pad_edges(edge_index_r0)
    src1, dst1 = _pad_edges(edge_index_r1)
    b1r = b1.reshape(1, -1)
    b2r = b2.reshape(1, -1)
    b3r = b3.reshape(1, -1)
    b4r = b4.reshape(1, -1)

    degp = _deg_call(dst0, dst1)
    h = _premlp_call(in_feat, W1, b1r, W2, b2r)

    dinv, g0 = _norm_call(h, degp)
    p = _prop_call(g0, src0, dst0)
    f1, g1 = _fuse_call(h, g0, p, dinv)
    q = _prop_call(g1, src0, dst0)
    h1, dinv1, g0b = _mid_call(h, f1, g1, q, dinv, W3, b3r, degp)

    p = _prop_call(g0b, src1, dst1)
    f1b, g1b = _fuse_call(h1, g0b, p, dinv1)
    q = _prop_call(g1b, src1, dst1)
    return _final_call(h1, f1b, g1b, q, dinv1, W3, b3r, W4, b4r)
